# baseline (speedup 1.0000x reference)
import functools

import jax
import jax.numpy as jnp
from jax import lax
from jax.experimental import pallas as pl
from jax.experimental.pallas import tpu as pltpu

N_ROIS = 200
CIN = 256
P = 14
R = 224
B = 8
RB = B * R
PAD = 16


def _head_kernel(x_ref, wc_ref, bc_ref, wt_ref, bt_ref, w5_ref, b5_ref, o_ref):
    x = x_ref[...].reshape(RB, CIN)

    rows = lax.broadcasted_iota(jnp.int32, (RB, 1), 0) % R
    w_idx = rows % P
    maskl = w_idx != 0
    maskr = w_idx != P - 1
    penalty = jnp.where(rows < P * P, 0.0, -1e30).astype(jnp.float32)

    zpad = jnp.zeros((PAD, CIN), jnp.bfloat16)

    def conv3x3_relu(x, li):
        zero = jnp.zeros((), jnp.bfloat16)
        ap = jnp.concatenate([zpad, x, zpad])
        apl = jnp.concatenate([zpad, jnp.where(maskl, x, zero), zpad])
        apr = jnp.concatenate([zpad, jnp.where(maskr, x, zero), zpad])
        acc = jnp.broadcast_to(bc_ref[li][None, :], (RB, CIN)) + penalty
        for t in range(9):
            ky, kx = t // 3, t % 3
            s = (ky - 1) * P + (kx - 1)
            src = apl if kx == 2 else (apr if kx == 0 else ap)
            acc = acc + jnp.dot(src[PAD + s:PAD + s + RB],
                                wc_ref[li, t],
                                preferred_element_type=jnp.float32)
        return jax.nn.relu(acc).astype(jnp.bfloat16)

    for li in range(4):
        x = conv3x3_relu(x, li)

    for t in range(4):
        p = jnp.dot(x, wt_ref[t], preferred_element_type=jnp.float32)
        p = jax.nn.relu(p + bt_ref[...]).astype(jnp.bfloat16)
        y = jnp.dot(p, w5_ref[...], preferred_element_type=jnp.float32)
        o_ref[:, t, :, :] = jax.nn.sigmoid(y + b5_ref[...]).reshape(B, R, 3)


def kernel(features, w1, b1, w2, b2, w3, b3, w4, b4, wt, bt, w5, b5):
    fx = jnp.transpose(features, (0, 2, 3, 1)).reshape(N_ROIS, P * P, CIN)
    fx = jnp.pad(fx, ((0, 0), (0, R - P * P), (0, 0))).astype(jnp.bfloat16)

    wc = jnp.stack([jnp.transpose(w, (2, 3, 1, 0)).reshape(9, CIN, CIN)
                    for w in (w1, w2, w3, w4)]).astype(jnp.bfloat16)
    bc = jnp.stack([b1, b2, b3, b4])
    wtm = jnp.transpose(wt, (2, 3, 0, 1)).reshape(4, CIN, CIN).astype(jnp.bfloat16)
    w5m = jnp.transpose(w5[:, :, 0, 0]).astype(jnp.bfloat16)

    out = pl.pallas_call(
        _head_kernel,
        grid=(N_ROIS // B,),
        in_specs=[
            pl.BlockSpec((B, R, CIN), lambda i: (i, 0, 0)),
            pl.BlockSpec((4, 9, CIN, CIN), lambda i: (0, 0, 0, 0)),
            pl.BlockSpec((4, CIN), lambda i: (0, 0)),
            pl.BlockSpec((4, CIN, CIN), lambda i: (0, 0, 0)),
            pl.BlockSpec((1, CIN), lambda i: (0, 0)),
            pl.BlockSpec((CIN, 3), lambda i: (0, 0)),
            pl.BlockSpec((1, 3), lambda i: (0, 0)),
        ],
        out_specs=pl.BlockSpec((B, 4, R, 3), lambda i: (i, 0, 0, 0)),
        out_shape=jax.ShapeDtypeStruct((N_ROIS, 4, R, 3), jnp.float32),
        compiler_params=pltpu.CompilerParams(
            dimension_semantics=("parallel",)),
    )(fx, wc, bc, wtm, bt[None, :], w5m, b5[None, :])

    o = out[:, :, :P * P, :].reshape(N_ROIS, 2, 2, P, P, 3)
    return o.transpose(0, 5, 3, 1, 4, 2).reshape(N_ROIS, 3, 2 * P, 2 * P)


# speedup vs baseline: 1.1149x; 1.1149x over previous
import jax
import jax.numpy as jnp
from jax import lax
from jax.experimental import pallas as pl
from jax.experimental.pallas import tpu as pltpu

N_ROIS = 200
CIN = 256
P = 14
R = 224
B = 8
RB = B * R
PAD = 16


def _head_kernel(x_ref, wc_ref, bc_ref, wt_ref, bt_ref, w5_ref, b5_ref, o_ref):
    x = x_ref[...].reshape(RB, CIN)

    rows = lax.broadcasted_iota(jnp.int32, (RB, 1), 0) % R
    w_idx = rows % P
    maskl = w_idx != 0
    maskr = w_idx != P - 1
    penalty = jnp.where(rows < P * P, 0.0, -1e30).astype(jnp.float32)

    zpad = jnp.zeros((PAD, CIN), jnp.bfloat16)
    zero = jnp.zeros((), jnp.bfloat16)

    def conv3x3_relu(x, li):
        ap = jnp.concatenate([zpad, x, zpad])
        apl = jnp.concatenate([zpad, jnp.where(maskl, x, zero), zpad])
        apr = jnp.concatenate([zpad, jnp.where(maskr, x, zero), zpad])
        cols = []
        for t in range(9):
            ky, kx = t // 3, t % 3
            s = (ky - 1) * P + (kx - 1)
            src = apl if kx == 2 else (apr if kx == 0 else ap)
            cols.append(src[PAD + s:PAD + s + RB])
        x9 = jnp.concatenate(cols, axis=1)
        acc = jnp.dot(x9, wc_ref[li], preferred_element_type=jnp.float32)
        acc = acc + (bc_ref[li][None, :] + penalty)
        return jax.nn.relu(acc).astype(jnp.bfloat16)

    for li in range(4):
        x = conv3x3_relu(x, li)

    up = jnp.dot(x, wt_ref[...], preferred_element_type=jnp.float32)
    up = jax.nn.relu(up + bt_ref[...]).astype(jnp.bfloat16)
    y = jnp.dot(up, w5_ref[...], preferred_element_type=jnp.float32)
    y = jax.nn.sigmoid(y + b5_ref[...])
    o_ref[...] = y.reshape(B, R, 12)


def kernel(features, w1, b1, w2, b2, w3, b3, w4, b4, wt, bt, w5, b5):
    fx = jnp.transpose(features, (0, 2, 3, 1)).reshape(N_ROIS, P * P, CIN)
    fx = jnp.pad(fx, ((0, 0), (0, R - P * P), (0, 0))).astype(jnp.bfloat16)

    wc = jnp.stack([jnp.transpose(w, (2, 3, 1, 0)).reshape(9 * CIN, CIN)
                    for w in (w1, w2, w3, w4)]).astype(jnp.bfloat16)
    bc = jnp.stack([b1, b2, b3, b4])
    wtm = jnp.transpose(wt, (2, 3, 0, 1)).reshape(4, CIN, CIN)
    wtm = jnp.concatenate([wtm[t] for t in range(4)], axis=1).astype(jnp.bfloat16)
    bt4 = jnp.tile(bt, 4)[None, :]
    w5m = jnp.transpose(w5[:, :, 0, 0])
    w5b = jnp.zeros((4 * CIN, 12), jnp.float32)
    for t in range(4):
        w5b = w5b.at[t * CIN:(t + 1) * CIN, t * 3:(t + 1) * 3].set(w5m)
    w5b = w5b.astype(jnp.bfloat16)
    b5b = jnp.tile(b5, 4)[None, :]

    out = pl.pallas_call(
        _head_kernel,
        grid=(N_ROIS // B,),
        in_specs=[
            pl.BlockSpec((B, R, CIN), lambda i: (i, 0, 0)),
            pl.BlockSpec((4, 9 * CIN, CIN), lambda i: (0, 0, 0)),
            pl.BlockSpec((4, CIN), lambda i: (0, 0)),
            pl.BlockSpec((CIN, 4 * CIN), lambda i: (0, 0)),
            pl.BlockSpec((1, 4 * CIN), lambda i: (0, 0)),
            pl.BlockSpec((4 * CIN, 12), lambda i: (0, 0)),
            pl.BlockSpec((1, 12), lambda i: (0, 0)),
        ],
        out_specs=pl.BlockSpec((B, R, 12), lambda i: (i, 0, 0)),
        out_shape=jax.ShapeDtypeStruct((N_ROIS, R, 12), jnp.float32),
        compiler_params=pltpu.CompilerParams(
            dimension_semantics=("parallel",)),
    )(fx, wc, bc, wtm, bt4, w5b, b5b)

    o = out[:, :P * P, :].reshape(N_ROIS, P, P, 2, 2, 3)
    return o.transpose(0, 5, 1, 3, 2, 4).reshape(N_ROIS, 3, 2 * P, 2 * P)
